# baseline (speedup 1.0000x reference)
import functools

import jax
import jax.numpy as jnp
from jax import lax
from jax.experimental import pallas as pl
from jax.experimental.pallas import tpu as pltpu
from jax.experimental.pallas import tpu_sc as plsc

N = 10000
E = 320000
D = 128
NC, NS = 2, 16
NW = NC * NS
CH = 128
NCH = 80
EPW = NCH * CH
E_PAD = NW * EPW
N_ACC = 10016
RPS = 632
RPS_LAST = N_ACC - (NS - 1) * RPS


G = 4
NGRP = NCH // G
NBUF = 3


def _init_slices(body_fn, s):
    base = s * RPS

    @pl.when(s < NS - 1)
    def _full():
        for k in range(RPS // CH):
            body_fn(base + k * CH, CH)
        body_fn(base + (RPS // CH) * CH, RPS - (RPS // CH) * CH)

    @pl.when(s == NS - 1)
    def _last():
        for k in range(RPS_LAST // CH):
            body_fn(base + k * CH, CH)
        body_fn(base + (RPS_LAST // CH) * CH,
                RPS_LAST - (RPS_LAST // CH) * CH)


def _mesh():
    return plsc.VectorSubcoreMesh(core_axis_name="c", subcore_axis_name="s",
                                  num_cores=NC, num_subcores=NS)


def _sc_agg_body(T, SRC, DST, P, src_v, dst_v, buf0, buf1, buf2, acc,
                 sem0, sem1, sem2):
    c = lax.axis_index("c")
    s = lax.axis_index("s")
    w = s * NC + c
    bufs = (buf0, buf1, buf2)
    sems = (sem0, sem1, sem2)
    zv = jnp.zeros((16,), jnp.float32)

    def zrow(i, _):
        for k in range(D // 16):
            buf0[i, pl.ds(k * 16, 16)] = zv
        return 0
    lax.fori_loop(0, CH, zrow, 0)

    base = s * RPS
    _init_slices(
        lambda off, n: pltpu.sync_copy(buf0.at[pl.ds(0, n)],
                                       acc.at[pl.ds(off, n)]), s)

    plsc.subcore_barrier()

    def group(g, _):
        pltpu.sync_copy(SRC.at[w, pl.ds(g * G, G)], src_v)
        pltpu.sync_copy(DST.at[w, pl.ds(g * G, G)], dst_v)
        for j in range(min(NBUF, G)):
            pltpu.async_copy(T.at[src_v.at[j]], bufs[j % NBUF], sems[j % NBUF])
        for j in range(G):
            pltpu.make_async_copy(T.at[src_v.at[j]],
                                  bufs[j % NBUF], sems[j % NBUF]).wait()
            pltpu.sync_copy(bufs[j % NBUF], acc.at[dst_v.at[j]], add=True)
            if j + NBUF < G:
                pltpu.async_copy(T.at[src_v.at[j + NBUF]],
                                 bufs[j % NBUF], sems[j % NBUF])
        return 0

    lax.fori_loop(0, NGRP, group, 0)

    plsc.subcore_barrier()

    @pl.when(s < NS - 1)
    def _out_full():
        pltpu.sync_copy(acc.at[pl.ds(base, RPS)], P.at[c, pl.ds(base, RPS)])

    @pl.when(s == NS - 1)
    def _out_last():
        pltpu.sync_copy(acc.at[pl.ds(base, RPS_LAST)],
                        P.at[c, pl.ds(base, RPS_LAST)])


_sc_agg = pl.kernel(
    _sc_agg_body,
    out_type=[jax.ShapeDtypeStruct((NC, N_ACC, D), jnp.float32)],
    mesh=_mesh(),
    scratch_types=[
        pltpu.VMEM((G, CH), jnp.int32),
        pltpu.VMEM((G, CH), jnp.int32),
        pltpu.VMEM((CH, D), jnp.float32),
        pltpu.VMEM((CH, D), jnp.float32),
        pltpu.VMEM((CH, D), jnp.float32),
        pltpu.VMEM_SHARED((N_ACC, D), jnp.float32),
        pltpu.SemaphoreType.DMA,
        pltpu.SemaphoreType.DMA,
        pltpu.SemaphoreType.DMA,
    ],
)


def _sc_deg_body(DST, DEGOUT, dst_v, ones_v, degsh):
    c = lax.axis_index("c")
    s = lax.axis_index("s")
    w = s * NC + c
    zv = jnp.zeros((16,), jnp.float32)
    ov = jnp.ones((16,), jnp.float32)

    def zrow(i, _):
        for k in range(D // 16):
            ones_v[i, pl.ds(k * 16, 16)] = zv
        return 0
    lax.fori_loop(0, CH, zrow, 0)

    base = s * RPS
    _init_slices(
        lambda off, n: pltpu.sync_copy(ones_v.at[pl.ds(0, n)],
                                       degsh.at[pl.ds(off, n)]), s)

    def orow(i, _):
        for k in range(D // 16):
            ones_v[i, pl.ds(k * 16, 16)] = ov
        return 0
    lax.fori_loop(0, CH, orow, 0)

    plsc.subcore_barrier()

    def group(g, _):
        pltpu.sync_copy(DST.at[w, pl.ds(g * G, G)], dst_v)
        for j in range(G):
            pltpu.sync_copy(ones_v, degsh.at[dst_v.at[j]], add=True)
        return 0

    lax.fori_loop(0, NGRP, group, 0)

    plsc.subcore_barrier()

    @pl.when(s < NS - 1)
    def _out_full():
        pltpu.sync_copy(degsh.at[pl.ds(base, RPS)],
                        DEGOUT.at[c, pl.ds(base, RPS)])

    @pl.when(s == NS - 1)
    def _out_last():
        pltpu.sync_copy(degsh.at[pl.ds(base, RPS_LAST)],
                        DEGOUT.at[c, pl.ds(base, RPS_LAST)])


_sc_deg = pl.kernel(
    _sc_deg_body,
    out_type=[jax.ShapeDtypeStruct((NC, N_ACC, D), jnp.float32)],
    mesh=_mesh(),
    scratch_types=[
        pltpu.VMEM((G, CH), jnp.int32),
        pltpu.VMEM((CH, D), jnp.float32),
        pltpu.VMEM_SHARED((N_ACC, D), jnp.float32),
    ],
)


def _make_combine(relu):
    BM = 1000

    def body(x_ref, p0, p1, d0, d1, ws, wn, b, o_ref):
        xb = x_ref[...]
        deg = d0[:, 0:1] + d1[:, 0:1] + 1.0
        mean = (p0[...] + p1[...] + xb) / deg
        out = jnp.dot(xb, ws[...], preferred_element_type=jnp.float32)
        out = out + jnp.dot(mean, wn[...], preferred_element_type=jnp.float32)
        out = out + b[...]
        if relu:
            out = jnp.maximum(out, 0.0)
        o_ref[...] = out

    row = lambda i: (i, 0)
    fixed = lambda i: (0, 0)
    return pl.pallas_call(
        body,
        grid=(N // BM,),
        in_specs=[
            pl.BlockSpec((BM, D), row),
            pl.BlockSpec((BM, D), row),
            pl.BlockSpec((BM, D), row),
            pl.BlockSpec((BM, D), row),
            pl.BlockSpec((BM, D), row),
            pl.BlockSpec((D, D), fixed),
            pl.BlockSpec((D, D), fixed),
            pl.BlockSpec((1, D), fixed),
        ],
        out_specs=pl.BlockSpec((BM, D), row),
        out_shape=jax.ShapeDtypeStruct((N, D), jnp.float32),
    )


_combine_relu = _make_combine(True)
_combine_lin = _make_combine(False)


def kernel(x, edge_index, W_self1, W_neigh1, b1, W_self2, W_neigh2, b2):
    ei = edge_index.astype(jnp.int32)
    npad = E_PAD - E
    src_p = jnp.concatenate(
        [ei[0], jnp.zeros((npad,), jnp.int32)]).reshape(NW, NCH, CH)
    dst_p = jnp.concatenate(
        [ei[1], N + (jnp.arange(npad, dtype=jnp.int32) % (N_ACC - N))]
    ).reshape(NW, NCH, CH)

    (DEG,) = _sc_deg(dst_p)
    (P,) = _sc_agg(x, src_p, dst_p)
    h = _combine_relu(x, P[0], P[1], DEG[0], DEG[1],
                      W_self1, W_neigh1, b1.reshape(1, D))
    (Q,) = _sc_agg(h, src_p, dst_p)
    out = _combine_lin(h, Q[0], Q[1], DEG[0], DEG[1],
                       W_self2, W_neigh2, b2.reshape(1, D))
    return out


# speedup vs baseline: 1.0444x; 1.0444x over previous
import functools

import jax
import jax.numpy as jnp
from jax import lax
from jax.experimental import pallas as pl
from jax.experimental.pallas import tpu as pltpu
from jax.experimental.pallas import tpu_sc as plsc

N = 10000
E = 320000
D = 128
NC, NS = 2, 16
NW = NC * NS
CH = 128
NCH = 80
EPW = NCH * CH
E_PAD = NW * EPW
N_ACC = 10016
RPS = 632
RPS_LAST = N_ACC - (NS - 1) * RPS


G = 8
GD = 4
NGRP = NCH // G
NBUF = 3


def _init_slices(body_fn, s):
    base = s * RPS

    @pl.when(s < NS - 1)
    def _full():
        for k in range(RPS // CH):
            body_fn(base + k * CH, CH)
        body_fn(base + (RPS // CH) * CH, RPS - (RPS // CH) * CH)

    @pl.when(s == NS - 1)
    def _last():
        for k in range(RPS_LAST // CH):
            body_fn(base + k * CH, CH)
        body_fn(base + (RPS_LAST // CH) * CH,
                RPS_LAST - (RPS_LAST // CH) * CH)


def _mesh():
    return plsc.VectorSubcoreMesh(core_axis_name="c", subcore_axis_name="s",
                                  num_cores=NC, num_subcores=NS)


def _sc_agg_body(T, SRC, DST, P, src_v, dst_v, buf0, buf1, buf2, acc,
                 sem0, sem1, sem2):
    c = lax.axis_index("c")
    s = lax.axis_index("s")
    w = s * NC + c
    bufs = (buf0, buf1, buf2)
    sems = (sem0, sem1, sem2)
    zv = jnp.zeros((16,), jnp.float32)

    def zrow(i, _):
        for k in range(D // 16):
            buf0[i, pl.ds(k * 16, 16)] = zv
        return 0
    lax.fori_loop(0, CH, zrow, 0)

    base = s * RPS
    _init_slices(
        lambda off, n: pltpu.sync_copy(buf0.at[pl.ds(0, n)],
                                       acc.at[pl.ds(off, n)]), s)

    plsc.subcore_barrier()

    def group(g, _):
        pltpu.sync_copy(SRC.at[w, pl.ds(g * G, G)], src_v)
        for j in range(min(NBUF, G)):
            pltpu.async_copy(T.at[src_v.at[j]], bufs[j % NBUF], sems[j % NBUF])
        for j in range(G):
            if j % GD == 0:
                pltpu.sync_copy(DST.at[w, pl.ds(g * G + j, GD)], dst_v)
            pltpu.make_async_copy(T.at[src_v.at[j]],
                                  bufs[j % NBUF], sems[j % NBUF]).wait()
            pltpu.sync_copy(bufs[j % NBUF], acc.at[dst_v.at[j % GD]], add=True)
            if j + NBUF < G:
                pltpu.async_copy(T.at[src_v.at[j + NBUF]],
                                 bufs[j % NBUF], sems[j % NBUF])
        return 0

    lax.fori_loop(0, NGRP, group, 0)

    plsc.subcore_barrier()

    @pl.when(s < NS - 1)
    def _out_full():
        pltpu.sync_copy(acc.at[pl.ds(base, RPS)], P.at[c, pl.ds(base, RPS)])

    @pl.when(s == NS - 1)
    def _out_last():
        pltpu.sync_copy(acc.at[pl.ds(base, RPS_LAST)],
                        P.at[c, pl.ds(base, RPS_LAST)])


_sc_agg = pl.kernel(
    _sc_agg_body,
    out_type=[jax.ShapeDtypeStruct((NC, N_ACC, D), jnp.float32)],
    mesh=_mesh(),
    scratch_types=[
        pltpu.VMEM((G, CH), jnp.int32),
        pltpu.VMEM((GD, CH), jnp.int32),
        pltpu.VMEM((CH, D), jnp.float32),
        pltpu.VMEM((CH, D), jnp.float32),
        pltpu.VMEM((CH, D), jnp.float32),
        pltpu.VMEM_SHARED((N_ACC, D), jnp.float32),
        pltpu.SemaphoreType.DMA,
        pltpu.SemaphoreType.DMA,
        pltpu.SemaphoreType.DMA,
    ],
)


def _sc_deg_body(DST, DEGOUT, dst_v, ones_v, degsh):
    c = lax.axis_index("c")
    s = lax.axis_index("s")
    w = s * NC + c
    zv = jnp.zeros((16,), jnp.float32)
    ov = jnp.ones((16,), jnp.float32)

    def zrow(i, _):
        for k in range(D // 16):
            ones_v[i, pl.ds(k * 16, 16)] = zv
        return 0
    lax.fori_loop(0, CH, zrow, 0)

    base = s * RPS
    _init_slices(
        lambda off, n: pltpu.sync_copy(ones_v.at[pl.ds(0, n)],
                                       degsh.at[pl.ds(off, n)]), s)

    def orow(i, _):
        for k in range(D // 16):
            ones_v[i, pl.ds(k * 16, 16)] = ov
        return 0
    lax.fori_loop(0, CH, orow, 0)

    plsc.subcore_barrier()

    def group(g, _):
        pltpu.sync_copy(DST.at[w, pl.ds(g * G, G)], dst_v)
        for j in range(G):
            pltpu.sync_copy(ones_v, degsh.at[dst_v.at[j]], add=True)
        return 0

    lax.fori_loop(0, NGRP, group, 0)

    plsc.subcore_barrier()

    @pl.when(s < NS - 1)
    def _out_full():
        pltpu.sync_copy(degsh.at[pl.ds(base, RPS)],
                        DEGOUT.at[c, pl.ds(base, RPS)])

    @pl.when(s == NS - 1)
    def _out_last():
        pltpu.sync_copy(degsh.at[pl.ds(base, RPS_LAST)],
                        DEGOUT.at[c, pl.ds(base, RPS_LAST)])


_sc_deg = pl.kernel(
    _sc_deg_body,
    out_type=[jax.ShapeDtypeStruct((NC, N_ACC, D), jnp.float32)],
    mesh=_mesh(),
    scratch_types=[
        pltpu.VMEM((G, CH), jnp.int32),
        pltpu.VMEM((CH, D), jnp.float32),
        pltpu.VMEM_SHARED((N_ACC, D), jnp.float32),
    ],
)


def _make_combine(relu):
    BM = 1000

    def body(x_ref, p0, p1, d0, d1, ws, wn, b, o_ref):
        xb = x_ref[...]
        deg = d0[:, 0:1] + d1[:, 0:1] + 1.0
        mean = (p0[...] + p1[...] + xb) / deg
        out = jnp.dot(xb, ws[...], preferred_element_type=jnp.float32)
        out = out + jnp.dot(mean, wn[...], preferred_element_type=jnp.float32)
        out = out + b[...]
        if relu:
            out = jnp.maximum(out, 0.0)
        o_ref[...] = out

    row = lambda i: (i, 0)
    fixed = lambda i: (0, 0)
    return pl.pallas_call(
        body,
        grid=(N // BM,),
        in_specs=[
            pl.BlockSpec((BM, D), row),
            pl.BlockSpec((BM, D), row),
            pl.BlockSpec((BM, D), row),
            pl.BlockSpec((BM, D), row),
            pl.BlockSpec((BM, D), row),
            pl.BlockSpec((D, D), fixed),
            pl.BlockSpec((D, D), fixed),
            pl.BlockSpec((1, D), fixed),
        ],
        out_specs=pl.BlockSpec((BM, D), row),
        out_shape=jax.ShapeDtypeStruct((N, D), jnp.float32),
    )


_combine_relu = _make_combine(True)
_combine_lin = _make_combine(False)


def kernel(x, edge_index, W_self1, W_neigh1, b1, W_self2, W_neigh2, b2):
    ei = edge_index.astype(jnp.int32)
    npad = E_PAD - E
    src_p = jnp.concatenate(
        [ei[0], jnp.zeros((npad,), jnp.int32)]).reshape(NW, NCH, CH)
    dst_p = jnp.concatenate(
        [ei[1], N + (jnp.arange(npad, dtype=jnp.int32) % (N_ACC - N))]
    ).reshape(NW, NCH, CH)

    (DEG,) = _sc_deg(dst_p)
    (P,) = _sc_agg(x, src_p, dst_p)
    h = _combine_relu(x, P[0], P[1], DEG[0], DEG[1],
                      W_self1, W_neigh1, b1.reshape(1, D))
    (Q,) = _sc_agg(h, src_p, dst_p)
    out = _combine_lin(h, Q[0], Q[1], DEG[0], DEG[1],
                       W_self2, W_neigh2, b2.reshape(1, D))
    return out


# speedup vs baseline: 3.2310x; 3.0936x over previous
import functools

import jax
import jax.numpy as jnp
import numpy as np
from jax import lax
from jax.experimental import pallas as pl
from jax.experimental.pallas import tpu as pltpu
from jax.experimental.pallas import tpu_sc as plsc

N = 10000
E = 320000
D = 128
NC, NS = 2, 16
NW = NC * NS
CH = 128
NCH = 80
EPW = NCH * CH
E_PAD = NW * EPW
N_ACC = 10016
RPS = 632
RPS_LAST = N_ACC - (NS - 1) * RPS


G = 8
GD = 4
NGRP = NCH // G
NBUF = 3


def _init_slices(body_fn, s):
    base = s * RPS

    @pl.when(s < NS - 1)
    def _full():
        for k in range(RPS // CH):
            body_fn(base + k * CH, CH)
        body_fn(base + (RPS // CH) * CH, RPS - (RPS // CH) * CH)

    @pl.when(s == NS - 1)
    def _last():
        for k in range(RPS_LAST // CH):
            body_fn(base + k * CH, CH)
        body_fn(base + (RPS_LAST // CH) * CH,
                RPS_LAST - (RPS_LAST // CH) * CH)


def _mesh():
    return plsc.VectorSubcoreMesh(core_axis_name="c", subcore_axis_name="s",
                                  num_cores=NC, num_subcores=NS)


def _sc_agg_body(T, SRC, DST, P, src_v, dst_v, buf0, buf1, buf2, acc,
                 sem0, sem1, sem2):
    c = lax.axis_index("c")
    s = lax.axis_index("s")
    w = s * NC + c
    bufs = (buf0, buf1, buf2)
    sems = (sem0, sem1, sem2)
    zv = jnp.zeros((16,), jnp.float32)

    def zrow(i, _):
        for k in range(D // 16):
            buf0[i, pl.ds(k * 16, 16)] = zv
        return 0
    lax.fori_loop(0, CH, zrow, 0)

    base = s * RPS
    _init_slices(
        lambda off, n: pltpu.sync_copy(buf0.at[pl.ds(0, n)],
                                       acc.at[pl.ds(off, n)]), s)

    plsc.subcore_barrier()

    def group(g, _):
        pltpu.sync_copy(SRC.at[w, pl.ds(g * G, G)], src_v)
        for j in range(min(NBUF, G)):
            pltpu.async_copy(T.at[src_v.at[j]], bufs[j % NBUF], sems[j % NBUF])
        for j in range(G):
            if j % GD == 0:
                pltpu.sync_copy(DST.at[w, pl.ds(g * G + j, GD)], dst_v)
            pltpu.make_async_copy(T.at[src_v.at[j]],
                                  bufs[j % NBUF], sems[j % NBUF]).wait()
            pltpu.sync_copy(bufs[j % NBUF], acc.at[dst_v.at[j % GD]], add=True)
            if j + NBUF < G:
                pltpu.async_copy(T.at[src_v.at[j + NBUF]],
                                 bufs[j % NBUF], sems[j % NBUF])
        return 0

    lax.fori_loop(0, NGRP, group, 0)

    plsc.subcore_barrier()

    @pl.when(s < NS - 1)
    def _out_full():
        pltpu.sync_copy(acc.at[pl.ds(base, RPS)], P.at[c, pl.ds(base, RPS)])

    @pl.when(s == NS - 1)
    def _out_last():
        pltpu.sync_copy(acc.at[pl.ds(base, RPS_LAST)],
                        P.at[c, pl.ds(base, RPS_LAST)])


_sc_agg = pl.kernel(
    _sc_agg_body,
    out_type=[jax.ShapeDtypeStruct((NC, N_ACC, D), jnp.float32)],
    mesh=_mesh(),
    scratch_types=[
        pltpu.VMEM((G, CH), jnp.int32),
        pltpu.VMEM((GD, CH), jnp.int32),
        pltpu.VMEM((CH, D), jnp.float32),
        pltpu.VMEM((CH, D), jnp.float32),
        pltpu.VMEM((CH, D), jnp.float32),
        pltpu.VMEM_SHARED((N_ACC, D), jnp.float32),
        pltpu.SemaphoreType.DMA,
        pltpu.SemaphoreType.DMA,
        pltpu.SemaphoreType.DMA,
    ],
)


def _sc_deg_body(DST, DEGOUT, dst_v, ones_v, degsh):
    c = lax.axis_index("c")
    s = lax.axis_index("s")
    w = s * NC + c
    zv = jnp.zeros((16,), jnp.float32)
    ov = jnp.ones((16,), jnp.float32)

    def zrow(i, _):
        for k in range(D // 16):
            ones_v[i, pl.ds(k * 16, 16)] = zv
        return 0
    lax.fori_loop(0, CH, zrow, 0)

    base = s * RPS
    _init_slices(
        lambda off, n: pltpu.sync_copy(ones_v.at[pl.ds(0, n)],
                                       degsh.at[pl.ds(off, n)]), s)

    def orow(i, _):
        for k in range(D // 16):
            ones_v[i, pl.ds(k * 16, 16)] = ov
        return 0
    lax.fori_loop(0, CH, orow, 0)

    plsc.subcore_barrier()

    def group(g, _):
        pltpu.sync_copy(DST.at[w, pl.ds(g * G, G)], dst_v)
        for j in range(G):
            pltpu.sync_copy(ones_v, degsh.at[dst_v.at[j]], add=True)
        return 0

    lax.fori_loop(0, NGRP, group, 0)

    plsc.subcore_barrier()

    @pl.when(s < NS - 1)
    def _out_full():
        pltpu.sync_copy(degsh.at[pl.ds(base, RPS)],
                        DEGOUT.at[c, pl.ds(base, RPS)])

    @pl.when(s == NS - 1)
    def _out_last():
        pltpu.sync_copy(degsh.at[pl.ds(base, RPS_LAST)],
                        DEGOUT.at[c, pl.ds(base, RPS_LAST)])


_sc_deg = pl.kernel(
    _sc_deg_body,
    out_type=[jax.ShapeDtypeStruct((NC, N_ACC, D), jnp.float32)],
    mesh=_mesh(),
    scratch_types=[
        pltpu.VMEM((G, CH), jnp.int32),
        pltpu.VMEM((CH, D), jnp.float32),
        pltpu.VMEM_SHARED((N_ACC, D), jnp.float32),
    ],
)


def _make_combine(relu):
    BM = 1000

    def body(x_ref, p0, p1, d0, d1, ws, wn, b, o_ref):
        xb = x_ref[...]
        deg = d0[:, 0:1] + d1[:, 0:1] + 1.0
        mean = (p0[...] + p1[...] + xb) / deg
        out = jnp.dot(xb, ws[...], preferred_element_type=jnp.float32)
        out = out + jnp.dot(mean, wn[...], preferred_element_type=jnp.float32)
        out = out + b[...]
        if relu:
            out = jnp.maximum(out, 0.0)
        o_ref[...] = out

    row = lambda i: (i, 0)
    fixed = lambda i: (0, 0)
    return pl.pallas_call(
        body,
        grid=(N // BM,),
        in_specs=[
            pl.BlockSpec((BM, D), row),
            pl.BlockSpec((BM, D), row),
            pl.BlockSpec((BM, D), row),
            pl.BlockSpec((BM, D), row),
            pl.BlockSpec((BM, D), row),
            pl.BlockSpec((D, D), fixed),
            pl.BlockSpec((D, D), fixed),
            pl.BlockSpec((1, D), fixed),
        ],
        out_specs=pl.BlockSpec((BM, D), row),
        out_shape=jax.ShapeDtypeStruct((N, D), jnp.float32),
    )


_combine_relu = _make_combine(True)
_combine_lin = _make_combine(False)


def kernel(x, edge_index, W_self1, W_neigh1, b1, W_self2, W_neigh2, b2):
    ei = edge_index.astype(jnp.int32)
    npad = E_PAD - E
    pad_src = (jnp.arange(npad, dtype=jnp.int32) * 97) % N
    src_p = jnp.concatenate([ei[0], pad_src]).reshape(NW, NCH, CH)
    dst_p = jnp.concatenate(
        [ei[1], N + (jnp.arange(npad, dtype=jnp.int32) % (N_ACC - N))]
    ).reshape(NW, NCH, CH)

    (DEG,) = _sc_deg(dst_p)
    (P,) = _sc_agg(x, src_p, dst_p)
    h = _combine_relu(x, P[0], P[1], DEG[0], DEG[1],
                      W_self1, W_neigh1, b1.reshape(1, D))
    (Q,) = _sc_agg(h, src_p, dst_p)
    out = _combine_lin(h, Q[0], Q[1], DEG[0], DEG[1],
                       W_self2, W_neigh2, b2.reshape(1, D))
    return out


# speedup vs baseline: 3.2377x; 1.0021x over previous
import functools

import jax
import jax.numpy as jnp
import numpy as np
from jax import lax
from jax.experimental import pallas as pl
from jax.experimental.pallas import tpu as pltpu
from jax.experimental.pallas import tpu_sc as plsc

N = 10000
E = 320000
D = 128
NC, NS = 2, 16
NW = NC * NS
CH = 128
NCH = 80
EPW = NCH * CH
E_PAD = NW * EPW
N_ACC = 10016
RPS = 632
RPS_LAST = N_ACC - (NS - 1) * RPS


G = 8
GD = 4
NGRP = NCH // G
NBUF = 3


def _init_slices(body_fn, s):
    base = s * RPS

    @pl.when(s < NS - 1)
    def _full():
        for k in range(RPS // CH):
            body_fn(base + k * CH, CH)
        body_fn(base + (RPS // CH) * CH, RPS - (RPS // CH) * CH)

    @pl.when(s == NS - 1)
    def _last():
        for k in range(RPS_LAST // CH):
            body_fn(base + k * CH, CH)
        body_fn(base + (RPS_LAST // CH) * CH,
                RPS_LAST - (RPS_LAST // CH) * CH)


def _mesh():
    return plsc.VectorSubcoreMesh(core_axis_name="c", subcore_axis_name="s",
                                  num_cores=NC, num_subcores=NS)


def _sc_agg_body(T, SRC, DST, P, src_v, dst_v, buf0, buf1, buf2, acc,
                 sem0, sem1, sem2, ssem0, ssem1, ssem2):
    c = lax.axis_index("c")
    s = lax.axis_index("s")
    w = s * NC + c
    bufs = (buf0, buf1, buf2)
    sems = (sem0, sem1, sem2)
    ssems = (ssem0, ssem1, ssem2)
    zv = jnp.zeros((16,), jnp.float32)

    def zrow(i, _):
        for k in range(D // 16):
            buf0[i, pl.ds(k * 16, 16)] = zv
        return 0
    lax.fori_loop(0, CH, zrow, 0)

    base = s * RPS
    _init_slices(
        lambda off, n: pltpu.sync_copy(buf0.at[pl.ds(0, n)],
                                       acc.at[pl.ds(off, n)]), s)

    plsc.subcore_barrier()

    def group(g, _):
        pltpu.sync_copy(SRC.at[w, pl.ds(g * G, G)], src_v)
        for j in range(min(NBUF, G)):
            pltpu.async_copy(T.at[src_v.at[j]], bufs[j % NBUF], sems[j % NBUF])
        for j in range(G):
            if j > 0:
                jp = j - 1
                pltpu.make_async_copy(
                    bufs[jp % NBUF], acc.at[dst_v.at[jp % GD]],
                    ssems[jp % NBUF]).wait()
                if jp + NBUF < G:
                    pltpu.async_copy(T.at[src_v.at[jp + NBUF]],
                                     bufs[jp % NBUF], sems[jp % NBUF])
            if j % GD == 0:
                pltpu.sync_copy(DST.at[w, pl.ds(g * G + j, GD)], dst_v)
            pltpu.make_async_copy(T.at[src_v.at[j]],
                                  bufs[j % NBUF], sems[j % NBUF]).wait()
            pltpu.async_copy(bufs[j % NBUF], acc.at[dst_v.at[j % GD]],
                             ssems[j % NBUF], add=True)
        jl = G - 1
        pltpu.make_async_copy(bufs[jl % NBUF], acc.at[dst_v.at[jl % GD]],
                              ssems[jl % NBUF]).wait()
        return 0

    lax.fori_loop(0, NGRP, group, 0)

    plsc.subcore_barrier()

    @pl.when(s < NS - 1)
    def _out_full():
        pltpu.sync_copy(acc.at[pl.ds(base, RPS)], P.at[c, pl.ds(base, RPS)])

    @pl.when(s == NS - 1)
    def _out_last():
        pltpu.sync_copy(acc.at[pl.ds(base, RPS_LAST)],
                        P.at[c, pl.ds(base, RPS_LAST)])


_sc_agg = pl.kernel(
    _sc_agg_body,
    out_type=[jax.ShapeDtypeStruct((NC, N_ACC, D), jnp.float32)],
    mesh=_mesh(),
    scratch_types=[
        pltpu.VMEM((G, CH), jnp.int32),
        pltpu.VMEM((GD, CH), jnp.int32),
        pltpu.VMEM((CH, D), jnp.float32),
        pltpu.VMEM((CH, D), jnp.float32),
        pltpu.VMEM((CH, D), jnp.float32),
        pltpu.VMEM_SHARED((N_ACC, D), jnp.float32),
        pltpu.SemaphoreType.DMA,
        pltpu.SemaphoreType.DMA,
        pltpu.SemaphoreType.DMA,
        pltpu.SemaphoreType.DMA,
        pltpu.SemaphoreType.DMA,
        pltpu.SemaphoreType.DMA,
    ],
)


def _sc_deg_body(DST, DEGOUT, dst_v, ones_v, degsh):
    c = lax.axis_index("c")
    s = lax.axis_index("s")
    w = s * NC + c
    zv = jnp.zeros((16,), jnp.float32)
    ov = jnp.ones((16,), jnp.float32)

    def zrow(i, _):
        for k in range(D // 16):
            ones_v[i, pl.ds(k * 16, 16)] = zv
        return 0
    lax.fori_loop(0, CH, zrow, 0)

    base = s * RPS
    _init_slices(
        lambda off, n: pltpu.sync_copy(ones_v.at[pl.ds(0, n)],
                                       degsh.at[pl.ds(off, n)]), s)

    def orow(i, _):
        for k in range(D // 16):
            ones_v[i, pl.ds(k * 16, 16)] = ov
        return 0
    lax.fori_loop(0, CH, orow, 0)

    plsc.subcore_barrier()

    def group(g, _):
        pltpu.sync_copy(DST.at[w, pl.ds(g * G, G)], dst_v)
        for j in range(G):
            pltpu.sync_copy(ones_v, degsh.at[dst_v.at[j]], add=True)
        return 0

    lax.fori_loop(0, NGRP, group, 0)

    plsc.subcore_barrier()

    @pl.when(s < NS - 1)
    def _out_full():
        pltpu.sync_copy(degsh.at[pl.ds(base, RPS)],
                        DEGOUT.at[c, pl.ds(base, RPS)])

    @pl.when(s == NS - 1)
    def _out_last():
        pltpu.sync_copy(degsh.at[pl.ds(base, RPS_LAST)],
                        DEGOUT.at[c, pl.ds(base, RPS_LAST)])


_sc_deg = pl.kernel(
    _sc_deg_body,
    out_type=[jax.ShapeDtypeStruct((NC, N_ACC, D), jnp.float32)],
    mesh=_mesh(),
    scratch_types=[
        pltpu.VMEM((G, CH), jnp.int32),
        pltpu.VMEM((CH, D), jnp.float32),
        pltpu.VMEM_SHARED((N_ACC, D), jnp.float32),
    ],
)


def _make_combine(relu):
    BM = 1000

    def body(x_ref, p0, p1, d0, d1, ws, wn, b, o_ref):
        xb = x_ref[...]
        deg = d0[:, 0:1] + d1[:, 0:1] + 1.0
        mean = (p0[...] + p1[...] + xb) / deg
        out = jnp.dot(xb, ws[...], preferred_element_type=jnp.float32)
        out = out + jnp.dot(mean, wn[...], preferred_element_type=jnp.float32)
        out = out + b[...]
        if relu:
            out = jnp.maximum(out, 0.0)
        o_ref[...] = out

    row = lambda i: (i, 0)
    fixed = lambda i: (0, 0)
    return pl.pallas_call(
        body,
        grid=(N // BM,),
        in_specs=[
            pl.BlockSpec((BM, D), row),
            pl.BlockSpec((BM, D), row),
            pl.BlockSpec((BM, D), row),
            pl.BlockSpec((BM, D), row),
            pl.BlockSpec((BM, D), row),
            pl.BlockSpec((D, D), fixed),
            pl.BlockSpec((D, D), fixed),
            pl.BlockSpec((1, D), fixed),
        ],
        out_specs=pl.BlockSpec((BM, D), row),
        out_shape=jax.ShapeDtypeStruct((N, D), jnp.float32),
    )


_combine_relu = _make_combine(True)
_combine_lin = _make_combine(False)


def kernel(x, edge_index, W_self1, W_neigh1, b1, W_self2, W_neigh2, b2):
    ei = edge_index.astype(jnp.int32)
    npad = E_PAD - E
    pad_src = (jnp.arange(npad, dtype=jnp.int32) * 97) % N
    src_p = jnp.concatenate([ei[0], pad_src]).reshape(NW, NCH, CH)
    dst_p = jnp.concatenate(
        [ei[1], N + (jnp.arange(npad, dtype=jnp.int32) % (N_ACC - N))]
    ).reshape(NW, NCH, CH)

    (DEG,) = _sc_deg(dst_p)
    (P,) = _sc_agg(x, src_p, dst_p)
    h = _combine_relu(x, P[0], P[1], DEG[0], DEG[1],
                      W_self1, W_neigh1, b1.reshape(1, D))
    (Q,) = _sc_agg(h, src_p, dst_p)
    out = _combine_lin(h, Q[0], Q[1], DEG[0], DEG[1],
                       W_self2, W_neigh2, b2.reshape(1, D))
    return out


# speedup vs baseline: 3.2789x; 1.0127x over previous
import functools

import jax
import jax.numpy as jnp
import numpy as np
from jax import lax
from jax.experimental import pallas as pl
from jax.experimental.pallas import tpu as pltpu
from jax.experimental.pallas import tpu_sc as plsc

N = 10000
E = 320000
D = 128
NC, NS = 2, 16
NW = NC * NS
CH = 128
NCH = 80
EPW = NCH * CH
E_PAD = NW * EPW
N_ACC = 10016
RPS = 632
RPS_LAST = N_ACC - (NS - 1) * RPS


G = 8
GD = 4
NGRP = NCH // G
NBUF = 3


def _init_slices(body_fn, s):
    base = s * RPS

    @pl.when(s < NS - 1)
    def _full():
        for k in range(RPS // CH):
            body_fn(base + k * CH, CH)
        body_fn(base + (RPS // CH) * CH, RPS - (RPS // CH) * CH)

    @pl.when(s == NS - 1)
    def _last():
        for k in range(RPS_LAST // CH):
            body_fn(base + k * CH, CH)
        body_fn(base + (RPS_LAST // CH) * CH,
                RPS_LAST - (RPS_LAST // CH) * CH)


def _mesh():
    return plsc.VectorSubcoreMesh(core_axis_name="c", subcore_axis_name="s",
                                  num_cores=NC, num_subcores=NS)


def _sc_agg_body(T, SRC, DST, P, src_v, dst_v, buf0, buf1, buf2, acc,
                 sem0, sem1, sem2, ssem0, ssem1, ssem2):
    c = lax.axis_index("c")
    s = lax.axis_index("s")
    w = s * NC + c
    bufs = (buf0, buf1, buf2)
    sems = (sem0, sem1, sem2)
    ssems = (ssem0, ssem1, ssem2)
    zv = jnp.zeros((16,), jnp.float32)

    def zrow(i, _):
        for k in range(D // 16):
            buf0[i, pl.ds(k * 16, 16)] = zv
        return 0
    lax.fori_loop(0, CH, zrow, 0)

    base = s * RPS
    _init_slices(
        lambda off, n: pltpu.sync_copy(buf0.at[pl.ds(0, n)],
                                       acc.at[pl.ds(off, n)]), s)

    plsc.subcore_barrier()

    def group(g, _):
        pltpu.sync_copy(SRC.at[w, pl.ds(g * G, G)], src_v)
        for j in range(min(NBUF, G)):
            pltpu.async_copy(T.at[src_v.at[j]], bufs[j % NBUF], sems[j % NBUF])
        for j in range(G):
            if j > 0:
                jp = j - 1
                pltpu.make_async_copy(
                    bufs[jp % NBUF], acc.at[dst_v.at[jp % GD]],
                    ssems[jp % NBUF]).wait()
                if jp + NBUF < G:
                    pltpu.async_copy(T.at[src_v.at[jp + NBUF]],
                                     bufs[jp % NBUF], sems[jp % NBUF])
            if j % GD == 0:
                pltpu.sync_copy(DST.at[w, pl.ds(g * G + j, GD)], dst_v)
            pltpu.make_async_copy(T.at[src_v.at[j]],
                                  bufs[j % NBUF], sems[j % NBUF]).wait()
            pltpu.async_copy(bufs[j % NBUF], acc.at[dst_v.at[j % GD]],
                             ssems[j % NBUF], add=True)
        jl = G - 1
        pltpu.make_async_copy(bufs[jl % NBUF], acc.at[dst_v.at[jl % GD]],
                              ssems[jl % NBUF]).wait()
        return 0

    lax.fori_loop(0, NGRP, group, 0)

    plsc.subcore_barrier()

    @pl.when(s < NS - 1)
    def _out_full():
        pltpu.sync_copy(acc.at[pl.ds(base, RPS)], P.at[c, pl.ds(base, RPS)])

    @pl.when(s == NS - 1)
    def _out_last():
        pltpu.sync_copy(acc.at[pl.ds(base, RPS_LAST)],
                        P.at[c, pl.ds(base, RPS_LAST)])


_sc_agg = pl.kernel(
    _sc_agg_body,
    out_type=[jax.ShapeDtypeStruct((NC, N_ACC, D), jnp.float32)],
    mesh=_mesh(),
    scratch_types=[
        pltpu.VMEM((G, CH), jnp.int32),
        pltpu.VMEM((GD, CH), jnp.int32),
        pltpu.VMEM((CH, D), jnp.float32),
        pltpu.VMEM((CH, D), jnp.float32),
        pltpu.VMEM((CH, D), jnp.float32),
        pltpu.VMEM_SHARED((N_ACC, D), jnp.float32),
        pltpu.SemaphoreType.DMA,
        pltpu.SemaphoreType.DMA,
        pltpu.SemaphoreType.DMA,
        pltpu.SemaphoreType.DMA,
        pltpu.SemaphoreType.DMA,
        pltpu.SemaphoreType.DMA,
    ],
)


def _sc_deg_body(DST, DEGOUT, dst_v, ones_v, degsh):
    c = lax.axis_index("c")
    s = lax.axis_index("s")
    w = s * NC + c
    zv = jnp.zeros((16,), jnp.float32)
    ov = jnp.ones((16,), jnp.float32)

    def zrow(i, _):
        for k in range(D // 16):
            ones_v[i, pl.ds(k * 16, 16)] = zv
        return 0
    lax.fori_loop(0, CH, zrow, 0)

    base = s * RPS
    _init_slices(
        lambda off, n: pltpu.sync_copy(ones_v.at[pl.ds(0, n)],
                                       degsh.at[pl.ds(off, n)]), s)

    def orow(i, _):
        for k in range(D // 16):
            ones_v[i, pl.ds(k * 16, 16)] = ov
        return 0
    lax.fori_loop(0, CH, orow, 0)

    plsc.subcore_barrier()

    def group(g, _):
        pltpu.sync_copy(DST.at[w, pl.ds(g * G, G)], dst_v)
        for j in range(G):
            pltpu.sync_copy(ones_v, degsh.at[dst_v.at[j]], add=True)
        return 0

    lax.fori_loop(0, NGRP, group, 0)

    plsc.subcore_barrier()

    @pl.when(s < NS - 1)
    def _out_full():
        pltpu.sync_copy(degsh.at[pl.ds(base, RPS)],
                        DEGOUT.at[c, pl.ds(base, RPS)])

    @pl.when(s == NS - 1)
    def _out_last():
        pltpu.sync_copy(degsh.at[pl.ds(base, RPS_LAST)],
                        DEGOUT.at[c, pl.ds(base, RPS_LAST)])


_sc_deg = pl.kernel(
    _sc_deg_body,
    out_type=[jax.ShapeDtypeStruct((NC, N_ACC, D), jnp.float32)],
    mesh=_mesh(),
    scratch_types=[
        pltpu.VMEM((G, CH), jnp.int32),
        pltpu.VMEM((CH, D), jnp.float32),
        pltpu.VMEM_SHARED((N_ACC, D), jnp.float32),
    ],
)


def _make_combine(relu):
    BM = 2000

    def body(x_ref, p0, p1, d0, d1, ws, wn, b, o_ref):
        xb = x_ref[...]
        deg = d0[:, 0:1] + d1[:, 0:1] + 1.0
        mean = (p0[...] + p1[...] + xb) / deg
        out = jnp.dot(xb, ws[...], preferred_element_type=jnp.float32)
        out = out + jnp.dot(mean, wn[...], preferred_element_type=jnp.float32)
        out = out + b[...]
        if relu:
            out = jnp.maximum(out, 0.0)
        o_ref[...] = out

    row = lambda i: (i, 0)
    fixed = lambda i: (0, 0)
    return pl.pallas_call(
        body,
        grid=(N // BM,),
        in_specs=[
            pl.BlockSpec((BM, D), row),
            pl.BlockSpec((BM, D), row),
            pl.BlockSpec((BM, D), row),
            pl.BlockSpec((BM, D), row),
            pl.BlockSpec((BM, D), row),
            pl.BlockSpec((D, D), fixed),
            pl.BlockSpec((D, D), fixed),
            pl.BlockSpec((1, D), fixed),
        ],
        out_specs=pl.BlockSpec((BM, D), row),
        out_shape=jax.ShapeDtypeStruct((N, D), jnp.float32),
    )


_combine_relu = _make_combine(True)
_combine_lin = _make_combine(False)


def kernel(x, edge_index, W_self1, W_neigh1, b1, W_self2, W_neigh2, b2):
    ei = edge_index.astype(jnp.int32)
    npad = E_PAD - E
    pad_src = (jnp.arange(npad, dtype=jnp.int32) * 97) % N
    src_p = jnp.concatenate([ei[0], pad_src]).reshape(NW, NCH, CH)
    dst_p = jnp.concatenate(
        [ei[1], N + (jnp.arange(npad, dtype=jnp.int32) % (N_ACC - N))]
    ).reshape(NW, NCH, CH)

    (DEG,) = _sc_deg(dst_p)
    (P,) = _sc_agg(x, src_p, dst_p)
    h = _combine_relu(x, P[0], P[1], DEG[0], DEG[1],
                      W_self1, W_neigh1, b1.reshape(1, D))
    (Q,) = _sc_agg(h, src_p, dst_p)
    out = _combine_lin(h, Q[0], Q[1], DEG[0], DEG[1],
                       W_self2, W_neigh2, b2.reshape(1, D))
    return out
